# Initial kernel scaffold; baseline (speedup 1.0000x reference)
import jax
import jax.numpy as jnp
from jax.experimental import pallas as pl


def kernel(node_feat, edge_index, W1l, b1, W1r, W2l, b2, W2r):
    raise NotImplementedError("write your pallas kernel here")


# speedup vs baseline: 7.2835x; 7.2835x over previous
import functools

import jax
import jax.numpy as jnp
from jax import lax
from jax.experimental import pallas as pl
from jax.experimental.pallas import tpu as pltpu
from jax.experimental.pallas import tpu_sc as plsc

N = 10000
D = 128
NC = 2
NT = 16
CH = 128
K = 80
G = 8
EPAD = NC * NT * K * CH
NPAD = 10240
RPT = NPAD // NT
BLK = 640


def _mesh():
    return plsc.VectorSubcoreMesh(core_axis_name="c", subcore_axis_name="s")


def _sc_agg_body(x_hbm, src_hbm, dst_hbm, zrow, acc_out,
                 src_v, dst_v, buf, acc_s, sem):
    cid = lax.axis_index("c")
    sid = lax.axis_index("s")
    base = sid * RPT

    pltpu.sync_copy(zrow, buf)
    for t in range(RPT // CH):
        pltpu.sync_copy(buf, acc_s.at[pl.ds(base + t * CH, CH)])
    plsc.subcore_barrier()

    def group(g, carry):
        gs = pl.ds(g * G, G)
        pltpu.sync_copy(src_hbm.at[cid, sid, gs], src_v)
        pltpu.sync_copy(dst_hbm.at[cid, sid, gs], dst_v)
        for j in range(G):
            pltpu.async_copy(x_hbm.at[src_v.at[j]], buf, sem).wait()
            pltpu.sync_copy(buf, acc_s.at[dst_v.at[j]], add=True)
        return carry

    lax.fori_loop(0, K // G, group, 0)
    plsc.subcore_barrier()

    for t in range(RPT // CH):
        sl = pl.ds(base + t * CH, CH)
        pltpu.sync_copy(acc_s.at[sl], buf)
        pltpu.sync_copy(buf, acc_out.at[cid, sl])


_sc_agg = pl.kernel(
    _sc_agg_body,
    out_type=jax.ShapeDtypeStruct((NC, NPAD, D), jnp.float32),
    mesh=_mesh(),
    scratch_types=[
        pltpu.VMEM((G, CH), jnp.int32),
        pltpu.VMEM((G, CH), jnp.int32),
        pltpu.VMEM((CH, D), jnp.float32),
        pltpu.VMEM_SHARED((NPAD, D), jnp.float32),
        pltpu.SemaphoreType.DMA,
    ],
)


def _sc_count_body(dst_hbm, zrow, ones, cnt_out, dst_v, buf, cnt_s):
    cid = lax.axis_index("c")
    sid = lax.axis_index("s")
    base = sid * RPT

    pltpu.sync_copy(zrow, buf)
    for t in range(RPT // CH):
        pltpu.sync_copy(buf, cnt_s.at[pl.ds(base + t * CH, CH)])
    pltpu.sync_copy(ones, buf)
    plsc.subcore_barrier()

    def group(g, carry):
        pltpu.sync_copy(dst_hbm.at[cid, sid, pl.ds(g * G, G)], dst_v)
        for j in range(G):
            pltpu.sync_copy(buf, cnt_s.at[dst_v.at[j]], add=True)
        return carry

    lax.fori_loop(0, K // G, group, 0)
    plsc.subcore_barrier()

    for t in range(RPT // CH):
        sl = pl.ds(base + t * CH, CH)
        pltpu.sync_copy(cnt_s.at[sl], buf)
        pltpu.sync_copy(buf, cnt_out.at[cid, sl])


_sc_count = pl.kernel(
    _sc_count_body,
    out_type=jax.ShapeDtypeStruct((NC, NPAD, D), jnp.float32),
    mesh=_mesh(),
    scratch_types=[
        pltpu.VMEM((G, CH), jnp.int32),
        pltpu.VMEM((CH, D), jnp.float32),
        pltpu.VMEM_SHARED((NPAD, D), jnp.float32),
    ],
)


def _dot_t(a, w):
    return lax.dot_general(a, w, (((1,), (1,)), ((), ())),
                           preferred_element_type=jnp.float32)


def _tc_layer_body(normalize, acc_ref, cnt_ref, x_ref, wl_ref, wr_ref, b_ref,
                   o_ref):
    agg = acc_ref[0] + acc_ref[1]
    cnt = cnt_ref[0, :, :1] + cnt_ref[1, :, :1]
    a = agg / jnp.maximum(cnt, 1.0)
    h = _dot_t(a, wl_ref[...]) + _dot_t(x_ref[...], wr_ref[...]) + b_ref[...]
    if normalize:
        nrm = jnp.sqrt(jnp.sum(h * h, axis=1, keepdims=True))
        h = h / jnp.maximum(nrm, 1e-12)
    o_ref[...] = jax.nn.sigmoid(h)


def _make_tc_layer(normalize):
    return pl.pallas_call(
        functools.partial(_tc_layer_body, normalize),
        grid=(NPAD // BLK,),
        in_specs=[
            pl.BlockSpec((NC, BLK, D), lambda i: (0, i, 0)),
            pl.BlockSpec((NC, BLK, D), lambda i: (0, i, 0)),
            pl.BlockSpec((BLK, D), lambda i: (i, 0)),
            pl.BlockSpec((D, D), lambda i: (0, 0)),
            pl.BlockSpec((D, D), lambda i: (0, 0)),
            pl.BlockSpec((1, D), lambda i: (0, 0)),
        ],
        out_specs=pl.BlockSpec((BLK, D), lambda i: (i, 0)),
        out_shape=jax.ShapeDtypeStruct((NPAD, D), jnp.float32),
    )


_tc_layer1 = _make_tc_layer(False)
_tc_layer2 = _make_tc_layer(True)


@jax.jit
def kernel(node_feat, edge_index, W1l, b1, W1r, W2l, b2, W2r):
    x_pad = jnp.concatenate(
        [node_feat, jnp.zeros((NPAD - N, D), jnp.float32)], axis=0)

    pad = EPAD - edge_index.shape[1]
    pad_idx = N + (jnp.arange(pad, dtype=jnp.int32) % NT)
    src_p = jnp.concatenate([edge_index[0], pad_idx]).reshape(NC, NT, K, CH)
    dst_p = jnp.concatenate([edge_index[1], pad_idx]).reshape(NC, NT, K, CH)

    zrow = jnp.zeros((CH, D), jnp.float32)
    ones = jnp.ones((CH, D), jnp.float32)

    cnt = _sc_count(dst_p, zrow, ones)
    acc1 = _sc_agg(x_pad, src_p, dst_p, zrow)
    x1 = _tc_layer1(acc1, cnt, x_pad, W1l, W1r, b1.reshape(1, D))
    acc2 = _sc_agg(x1, src_p, dst_p, zrow)
    x2 = _tc_layer2(acc2, cnt, x1, W2l, W2r, b2.reshape(1, D))
    return x2[:N]


# speedup vs baseline: 8.2456x; 1.1321x over previous
import functools

import jax
import jax.numpy as jnp
from jax import lax
from jax.experimental import pallas as pl
from jax.experimental.pallas import tpu as pltpu
from jax.experimental.pallas import tpu_sc as plsc

N = 10000
D = 128
NC = 2
NT = 16
CH = 64
K = 160
KG = 5
G = K // KG
EPAD = NC * NT * K * CH
NPAD = 10240
RPT = NPAD // NT
ZCH = 64
BLK = 640


def _mesh():
    return plsc.VectorSubcoreMesh(core_axis_name="c", subcore_axis_name="s")


def _sc_agg_body(x_hbm, src_hbm, dst_hbm, zrow, acc_out,
                 src_v, dst_v, bufa, bufb, acc_s, sema, semb):
    cid = lax.axis_index("c")
    sid = lax.axis_index("s")
    base = sid * RPT

    pltpu.sync_copy(zrow, bufa)
    for t in range(RPT // ZCH):
        pltpu.sync_copy(bufa, acc_s.at[pl.ds(base + t * ZCH, ZCH)])
    plsc.subcore_barrier()

    def group(g, carry):
        pltpu.sync_copy(src_hbm.at[cid, sid, g], src_v)
        pltpu.sync_copy(dst_hbm.at[cid, sid, g], dst_v)
        pltpu.async_copy(x_hbm.at[src_v.at[0]], bufa, sema)

        def pair(i, c2):
            j = 2 * i
            pltpu.async_copy(x_hbm.at[src_v.at[j + 1]], bufb, semb)
            pltpu.make_async_copy(x_hbm.at[src_v.at[j]], bufa, sema).wait()
            pltpu.sync_copy(bufa, acc_s.at[dst_v.at[j]], add=True)
            pltpu.async_copy(x_hbm.at[src_v.at[j + 2]], bufa, sema)
            pltpu.make_async_copy(x_hbm.at[src_v.at[j + 1]], bufb, semb).wait()
            pltpu.sync_copy(bufb, acc_s.at[dst_v.at[j + 1]], add=True)
            return c2

        lax.fori_loop(0, G // 2, pair, 0)
        pltpu.make_async_copy(x_hbm.at[src_v.at[G]], bufa, sema).wait()
        return carry

    lax.fori_loop(0, KG, group, 0)
    plsc.subcore_barrier()

    for t in range(RPT // ZCH):
        sl = pl.ds(base + t * ZCH, ZCH)
        pltpu.sync_copy(acc_s.at[sl], bufa)
        pltpu.sync_copy(bufa, acc_out.at[cid, sl])


_sc_agg = pl.kernel(
    _sc_agg_body,
    out_type=jax.ShapeDtypeStruct((NC, NPAD, D), jnp.float32),
    mesh=_mesh(),
    scratch_types=[
        pltpu.VMEM((G + 1, CH), jnp.int32),
        pltpu.VMEM((G, CH), jnp.int32),
        pltpu.VMEM((ZCH, D), jnp.float32),
        pltpu.VMEM((CH, D), jnp.float32),
        pltpu.VMEM_SHARED((NPAD, D), jnp.float32),
        pltpu.SemaphoreType.DMA,
        pltpu.SemaphoreType.DMA,
    ],
)


def _sc_count_body(dst_hbm, zrow, ones, cnt_out, dst_v, buf, cnt_s):
    cid = lax.axis_index("c")
    sid = lax.axis_index("s")
    base = sid * RPT

    pltpu.sync_copy(zrow, buf)
    for t in range(RPT // ZCH):
        pltpu.sync_copy(buf, cnt_s.at[pl.ds(base + t * ZCH, ZCH)])
    pltpu.sync_copy(ones, buf)
    pltpu.sync_copy(dst_hbm.at[cid, sid], dst_v)
    plsc.subcore_barrier()

    def step(j, carry):
        pltpu.sync_copy(buf.at[pl.ds(0, CH)], cnt_s.at[dst_v.at[j]], add=True)
        return carry

    lax.fori_loop(0, K, step, 0)
    plsc.subcore_barrier()

    for t in range(RPT // ZCH):
        sl = pl.ds(base + t * ZCH, ZCH)
        pltpu.sync_copy(cnt_s.at[sl], buf)
        pltpu.sync_copy(buf, cnt_out.at[cid, sl])


_sc_count = pl.kernel(
    _sc_count_body,
    out_type=jax.ShapeDtypeStruct((NC, NPAD, D), jnp.float32),
    mesh=_mesh(),
    scratch_types=[
        pltpu.VMEM((K, CH), jnp.int32),
        pltpu.VMEM((ZCH, D), jnp.float32),
        pltpu.VMEM_SHARED((NPAD, D), jnp.float32),
    ],
)


def _dot_t(a, w):
    return lax.dot_general(a, w, (((1,), (1,)), ((), ())),
                           preferred_element_type=jnp.float32)


def _tc_layer_body(normalize, acc_ref, cnt_ref, x_ref, wl_ref, wr_ref, b_ref,
                   o_ref):
    agg = acc_ref[0] + acc_ref[1]
    cnt = cnt_ref[0, :, :1] + cnt_ref[1, :, :1]
    a = agg / jnp.maximum(cnt, 1.0)
    h = _dot_t(a, wl_ref[...]) + _dot_t(x_ref[...], wr_ref[...]) + b_ref[...]
    if normalize:
        nrm = jnp.sqrt(jnp.sum(h * h, axis=1, keepdims=True))
        h = h / jnp.maximum(nrm, 1e-12)
    o_ref[...] = jax.nn.sigmoid(h)


def _make_tc_layer(normalize):
    return pl.pallas_call(
        functools.partial(_tc_layer_body, normalize),
        grid=(NPAD // BLK,),
        in_specs=[
            pl.BlockSpec((NC, BLK, D), lambda i: (0, i, 0)),
            pl.BlockSpec((NC, BLK, D), lambda i: (0, i, 0)),
            pl.BlockSpec((BLK, D), lambda i: (i, 0)),
            pl.BlockSpec((D, D), lambda i: (0, 0)),
            pl.BlockSpec((D, D), lambda i: (0, 0)),
            pl.BlockSpec((1, D), lambda i: (0, 0)),
        ],
        out_specs=pl.BlockSpec((BLK, D), lambda i: (i, 0)),
        out_shape=jax.ShapeDtypeStruct((NPAD, D), jnp.float32),
    )


_tc_layer1 = _make_tc_layer(False)
_tc_layer2 = _make_tc_layer(True)


@jax.jit
def kernel(node_feat, edge_index, W1l, b1, W1r, W2l, b2, W2r):
    x_pad = jnp.concatenate(
        [node_feat, jnp.zeros((NPAD - N, D), jnp.float32)], axis=0)

    pad = EPAD - edge_index.shape[1]
    pad_idx = N + (jnp.arange(pad, dtype=jnp.int32) % NT)
    src_p = jnp.concatenate([edge_index[0], pad_idx]).reshape(
        NC, NT, KG, G, CH)
    dst_p = jnp.concatenate([edge_index[1], pad_idx]).reshape(NC, NT, K, CH)
    xtra = jnp.broadcast_to(
        (N + (jnp.arange(CH, dtype=jnp.int32) % NT)).reshape(1, 1, 1, 1, CH),
        (NC, NT, KG, 1, CH))
    src_p = jnp.concatenate([src_p, xtra], axis=3)

    zrow = jnp.zeros((ZCH, D), jnp.float32)
    ones = jnp.ones((ZCH, D), jnp.float32)

    dst_g = dst_p.reshape(NC, NT, KG, G, CH)
    cnt = _sc_count(dst_p, zrow, ones)
    acc1 = _sc_agg(x_pad, src_p, dst_g, zrow)
    x1 = _tc_layer1(acc1, cnt, x_pad, W1l, W1r, b1.reshape(1, D))
    acc2 = _sc_agg(x1, src_p, dst_g, zrow)
    x2 = _tc_layer2(acc2, cnt, x1, W2l, W2r, b2.reshape(1, D))
    return x2[:N]
